# baseline (speedup 1.0000x reference)
import functools

import jax
import jax.numpy as jnp
from jax.experimental import pallas as pl
from jax.experimental.pallas import tpu as pltpu

_Q = 3
_K = 256
_D = 32
_B = 64
_N = 1024
_T = _B * _N
_HALVES = 4
_TH = _T // _HALVES
_BH = _B // _HALVES
_TBLK = 2048
_NBLK = _TH // _TBLK
_COUNT = _T * _D


def _rvq_block(xt_ref, cb_ref, cbt_ref, quant_ref, idx_ref, loss_ref):
    b = pl.program_id(0)

    @pl.when(b == 0)
    def _init():
        loss_ref[...] = jnp.zeros_like(loss_ref)

    rt = xt_ref[...]
    quant = jnp.zeros_like(rt)
    iota_k = jax.lax.broadcasted_iota(jnp.int32, (_K, _TBLK), 0)
    lane_iota = jax.lax.broadcasted_iota(jnp.int32, (1, 128), 1)
    loss_acc = loss_ref[...]
    for i in range(_Q):
        c = cb_ref[i]
        ct = cbt_ref[i]
        n_c = jnp.sum(c * c, axis=-1).reshape(_K, 1)
        sq = rt * rt
        h = sq[0:16, :] + sq[16:32, :]
        h = h[0:8, :] + h[8:16, :]
        s_r = jnp.sum(h, axis=0, keepdims=True)
        s = jax.lax.dot_general(
            c * -2.0, rt, (((1,), (0,)), ((), ())),
            precision=jax.lax.Precision.DEFAULT,
            preferred_element_type=jnp.float32)
        d = (s_r + s) + n_c
        m = jnp.min(d, axis=0, keepdims=True)
        idx = jnp.min(jnp.where(d == m, iota_k, _K), axis=0)
        idxb = jnp.broadcast_to(idx[None, :], (_D, _TBLK))
        ilow = jnp.bitwise_and(idxb, 127)
        q0 = jnp.take_along_axis(ct[:, 0:128], ilow, axis=1)
        q1 = jnp.take_along_axis(ct[:, 128:256], ilow, axis=1)
        q = jnp.where(idxb < 128, q0, q1)
        q_st = rt + (q - rt)
        sumsq = jnp.sum((q - rt) * (q - rt)) * (1.0 / _COUNT)
        loss_acc = loss_acc + jnp.where(lane_iota == i, sumsq, 0.0)
        quant = quant + q_st
        rt = rt - q_st
        idx_ref[0, i, :] = idx
    quant_ref[...] = quant
    loss_ref[...] = loss_acc


def _run_half(xh, codebooks, cbt):
    xt = xh.reshape(_TH, _D).T
    grid = (_NBLK,)
    return pl.pallas_call(
        _rvq_block,
        grid=grid,
        in_specs=[
            pl.BlockSpec((_D, _TBLK), lambda b: (0, b)),
            pl.BlockSpec((_Q, _K, _D), lambda b: (0, 0, 0)),
            pl.BlockSpec((_Q, _D, _K), lambda b: (0, 0, 0)),
        ],
        out_specs=[
            pl.BlockSpec((_D, _TBLK), lambda b: (0, b)),
            pl.BlockSpec((1, _Q, _TBLK), lambda b: (b, 0, 0)),
            pl.BlockSpec((1, 128), lambda b: (0, 0)),
        ],
        out_shape=[
            jax.ShapeDtypeStruct((_D, _TH), jnp.float32),
            jax.ShapeDtypeStruct((_NBLK, _Q, _TBLK), jnp.int32),
            jax.ShapeDtypeStruct((1, 128), jnp.float32),
        ],
        compiler_params=pltpu.CompilerParams(
            dimension_semantics=("arbitrary",),
        ),
    )(xt, codebooks, cbt)


@functools.partial(jax.jit, static_argnames=())
def kernel(x, codebooks):
    cbt = codebooks.transpose(0, 2, 1)
    quants, idxs, losses = [], [], []
    for hh in range(_HALVES):
        q, ix, ls = _run_half(x[hh * _BH:(hh + 1) * _BH], codebooks, cbt)
        quants.append(q.T.reshape(_BH, _N, _D))
        idxs.append(ix.transpose(0, 2, 1).reshape(_BH, _N, _Q))
        losses.append(ls)
    quantized = jnp.concatenate(quants, axis=0)
    indices = jnp.concatenate(idxs, axis=0)
    commit_loss = sum(losses)[0, :_Q]
    return (quantized, indices, commit_loss)


# speedup vs baseline: 1.1142x; 1.1142x over previous
import functools

import jax
import jax.numpy as jnp
from jax.experimental import pallas as pl
from jax.experimental.pallas import tpu as pltpu

_Q = 3
_K = 256
_D = 32
_B = 64
_N = 1024
_T = _B * _N
_HALVES = 2
_TH = _T // _HALVES
_BH = _B // _HALVES
_TBLK = 2048
_NBLK = _TH // _TBLK
_COUNT = _T * _D


def _rvq_block(xt_ref, cb_ref, cbt_ref, quant_ref, idx_ref, loss_ref):
    b = pl.program_id(0)

    @pl.when(b == 0)
    def _init():
        loss_ref[...] = jnp.zeros_like(loss_ref)

    rt = xt_ref[...]
    quant = jnp.zeros_like(rt)
    iota_k = jax.lax.broadcasted_iota(jnp.int32, (_K, _TBLK), 0)
    lane_iota = jax.lax.broadcasted_iota(jnp.int32, (1, 128), 1)
    loss_acc = loss_ref[...]
    for i in range(_Q):
        c = cb_ref[i]
        ct = cbt_ref[i]
        n_c = jnp.sum(c * c, axis=-1).reshape(_K, 1)
        sq = rt * rt
        h = sq[0:16, :] + sq[16:32, :]
        h = h[0:8, :] + h[8:16, :]
        s_r = jnp.sum(h, axis=0, keepdims=True)
        s = jax.lax.dot_general(
            c * -2.0, rt, (((1,), (0,)), ((), ())),
            precision=jax.lax.Precision.DEFAULT,
            preferred_element_type=jnp.float32)
        d = (s_r + s) + n_c
        m = jnp.min(d, axis=0, keepdims=True)
        idx = jnp.min(jnp.where(d == m, iota_k, _K), axis=0)
        idxb = jnp.broadcast_to(idx[None, :], (_D, _TBLK))
        ilow = jnp.bitwise_and(idxb, 127)
        q0 = jnp.take_along_axis(ct[:, 0:128], ilow, axis=1)
        q1 = jnp.take_along_axis(ct[:, 128:256], ilow, axis=1)
        q = jnp.where(idxb < 128, q0, q1)
        q_st = rt + (q - rt)
        sumsq = jnp.sum((q - rt) * (q - rt)) * (1.0 / _COUNT)
        loss_acc = loss_acc + jnp.where(lane_iota == i, sumsq, 0.0)
        quant = quant + q_st
        rt = rt - q_st
        idx_ref[0, i, :] = idx
    quant_ref[...] = quant
    loss_ref[...] = loss_acc


def _run_half(xh, codebooks, cbt):
    xt = xh.reshape(_TH, _D).T
    grid = (_NBLK,)
    return pl.pallas_call(
        _rvq_block,
        grid=grid,
        in_specs=[
            pl.BlockSpec((_D, _TBLK), lambda b: (0, b)),
            pl.BlockSpec((_Q, _K, _D), lambda b: (0, 0, 0)),
            pl.BlockSpec((_Q, _D, _K), lambda b: (0, 0, 0)),
        ],
        out_specs=[
            pl.BlockSpec((_D, _TBLK), lambda b: (0, b)),
            pl.BlockSpec((1, _Q, _TBLK), lambda b: (b, 0, 0)),
            pl.BlockSpec((1, 128), lambda b: (0, 0)),
        ],
        out_shape=[
            jax.ShapeDtypeStruct((_D, _TH), jnp.float32),
            jax.ShapeDtypeStruct((_NBLK, _Q, _TBLK), jnp.int32),
            jax.ShapeDtypeStruct((1, 128), jnp.float32),
        ],
        compiler_params=pltpu.CompilerParams(
            dimension_semantics=("arbitrary",),
        ),
    )(xt, codebooks, cbt)


@functools.partial(jax.jit, static_argnames=())
def kernel(x, codebooks):
    cbt = codebooks.transpose(0, 2, 1)
    quants, idxs, losses = [], [], []
    for hh in range(_HALVES):
        q, ix, ls = _run_half(x[hh * _BH:(hh + 1) * _BH], codebooks, cbt)
        quants.append(q.T.reshape(_BH, _N, _D))
        idxs.append(ix.transpose(0, 2, 1).reshape(_BH, _N, _Q))
        losses.append(ls)
    quantized = jnp.concatenate(quants, axis=0)
    indices = jnp.concatenate(idxs, axis=0)
    commit_loss = sum(losses)[0, :_Q]
    return (quantized, indices, commit_loss)
